# baseline (speedup 1.0000x reference)
import functools

import jax
import jax.numpy as jnp
from jax import lax
from jax.experimental import pallas as pl
from jax.experimental.pallas import tpu as pltpu
from jax.experimental.pallas import tpu_sc as plsc

N_NODES = 100000
N_GRAPHS = 100
ROWS_PER_GRAPH = 1000
PATH = 4
DIM = 128
FDIM = PATH * DIM
LANES = 16
NCHUNK = FDIM // LANES

CHUNK_ROWS = 100
CHUNK_ELEMS = CHUNK_ROWS * FDIM
CHUNKS_PER_GRAPH = ROWS_PER_GRAPH // CHUNK_ROWS

SC_GRAPHS = 40
TC_GRAPHS = N_GRAPHS - SC_GRAPHS

NUM_CORES = 2
NUM_SUBCORES = 16
G_PER_CORE = SC_GRAPHS // NUM_CORES
CHUNKS_PER_CORE = G_PER_CORE * CHUNKS_PER_GRAPH
MAX_SLOTS = 6


def _span_lo(s):
    return s * CHUNKS_PER_CORE // NUM_SUBCORES


def _owner(c):
    return (c * NUM_SUBCORES + NUM_SUBCORES - 1) // CHUNKS_PER_CORE


def _perm_base(c):
    p = c // (DIM // LANES)
    d0 = (c % (DIM // LANES)) * LANES
    return d0 * PATH + p


def _body(
    features_hbm,
    counts_hbm,
    out_hbm,
    buf0,
    buf1,
    fstage,
    f1,
    f2,
    staging_v,
    counts_v,
    parts_hbm,
    sem0,
    sem1,
):
    cid = lax.axis_index("c")
    sid = lax.axis_index("s")

    pltpu.sync_copy(counts_hbm, counts_v)

    zero = jnp.zeros((LANES,), jnp.float32)
    lane_iota = lax.iota(jnp.int32, LANES)

    c_lo = cid * CHUNKS_PER_CORE + sid * CHUNKS_PER_CORE // NUM_SUBCORES
    c_hi = cid * CHUNKS_PER_CORE + (sid + 1) * CHUNKS_PER_CORE // NUM_SUBCORES
    n_chunks = c_hi - c_lo
    first_graph = (c_lo - cid * CHUNKS_PER_CORE) // CHUNKS_PER_GRAPH
    part_base = (cid * NUM_SUBCORES + sid) * MAX_SLOTS * FDIM

    def chunk_copy(c, buf, sem):
        start = pl.multiple_of((c_lo + c) * CHUNK_ELEMS, 512)
        return pltpu.make_async_copy(
            features_hbm.at[pl.ds(start, CHUNK_ELEMS)], buf, sem
        )

    chunk_copy(0, buf0, sem0).start()
    chunk_copy(1, buf1, sem1).start()

    def step(c, buf, sem, acc):
        chunk_copy(c, buf, sem).wait()

        def row_body(r, a):
            off = r * FDIM
            return tuple(
                a[k] + buf[pl.ds(off + k * LANES, LANES)] for k in range(NCHUNK)
            )

        acc = lax.fori_loop(0, CHUNK_ROWS, row_body, acc)

        gc = c_lo + c
        is_flush = jnp.logical_or((gc + 1) % CHUNKS_PER_GRAPH == 0, c == n_chunks - 1)

        @pl.when(is_flush)
        def _():
            g_local = (gc - cid * CHUNKS_PER_CORE) // CHUNKS_PER_GRAPH
            slot = g_local - first_graph
            for k in range(NCHUNK):
                fstage[pl.ds(k * LANES, LANES)] = acc[k]
            dst = pl.multiple_of(part_base + slot * FDIM, 512)
            pltpu.sync_copy(fstage, parts_hbm.at[pl.ds(dst, FDIM)])

        @pl.when(c + 2 < n_chunks)
        def _():
            chunk_copy(c + 2, buf, sem).start()

        return tuple(jnp.where(is_flush, zero, a) for a in acc)

    def pair_body(i, acc):
        acc = step(2 * i, buf0, sem0, acc)
        acc = step(2 * i + 1, buf1, sem1, acc)
        return acc

    acc = lax.fori_loop(0, n_chunks // 2, pair_body, (zero,) * NCHUNK)

    @pl.when(n_chunks % 2 == 1)
    def _():
        step(n_chunks - 1, buf0, sem0, acc)

    plsc.subcore_barrier()

    gl_lo = sid * G_PER_CORE // NUM_SUBCORES
    gl_hi = (sid + 1) * G_PER_CORE // NUM_SUBCORES
    lane4 = lane_iota * PATH

    def do_out(gl, carry):
        c_first = gl * CHUNKS_PER_GRAPH
        c_last = c_first + CHUNKS_PER_GRAPH - 1
        s1 = _owner(c_first)
        s2 = _owner(c_last)
        slot1 = gl - _span_lo(s1) // CHUNKS_PER_GRAPH
        slot2 = gl - _span_lo(s2) // CHUNKS_PER_GRAPH
        src1 = pl.multiple_of(
            ((cid * NUM_SUBCORES + s1) * MAX_SLOTS + slot1) * FDIM, 512
        )
        src2 = pl.multiple_of(
            ((cid * NUM_SUBCORES + s2) * MAX_SLOTS + slot2) * FDIM, 512
        )
        pltpu.sync_copy(parts_hbm.at[pl.ds(src1, FDIM)], f1)
        pltpu.sync_copy(parts_hbm.at[pl.ds(src2, FDIM)], f2)
        m = jnp.broadcast_to((s1 != s2).astype(jnp.float32), (LANES,))
        g = cid * G_PER_CORE + gl
        cnt = counts_v[pl.ds(g, LANES)][0]
        cnt_v = jnp.broadcast_to(cnt.astype(jnp.float32), (LANES,))
        scale = 1.0 / jnp.maximum(cnt_v, 1.0)
        for k in range(NCHUNK):
            val = f1[pl.ds(k * LANES, LANES)] + m * f2[pl.ds(k * LANES, LANES)]
            plsc.store_scatter(staging_v, [lane4 + _perm_base(k)], val * scale)
        out_start = pl.multiple_of(g * FDIM, 512)
        pltpu.sync_copy(staging_v, out_hbm.at[pl.ds(out_start, FDIM)])
        return carry

    lax.fori_loop(gl_lo, gl_hi, do_out, 0)


def _tc_body(counts_smem, feat_ref, out_ref):
    g = pl.program_id(0) + SC_GRAPHS
    cnt = counts_smem[g]
    scale = 1.0 / jnp.maximum(cnt.astype(jnp.float32), 1.0)
    s = jnp.sum(feat_ref[...], axis=0)
    t = jnp.swapaxes(s, 0, 1)
    out_ref[...] = (t * scale).reshape(1, DIM, PATH)


@jax.jit
def kernel(features, n_nodes):

    f = features.reshape(N_NODES * FDIM)
    counts = jnp.pad(n_nodes, (0, 28))

    sc_run = pl.kernel(
        _body,
        out_type=jax.ShapeDtypeStruct((SC_GRAPHS * FDIM,), jnp.float32),
        mesh=plsc.VectorSubcoreMesh(core_axis_name="c", subcore_axis_name="s"),
        compiler_params=pltpu.CompilerParams(needs_layout_passes=False),
        scratch_types=[
            pltpu.VMEM((CHUNK_ELEMS,), jnp.float32),
            pltpu.VMEM((CHUNK_ELEMS,), jnp.float32),
            pltpu.VMEM((FDIM,), jnp.float32),
            pltpu.VMEM((FDIM,), jnp.float32),
            pltpu.VMEM((FDIM,), jnp.float32),
            pltpu.VMEM((FDIM,), jnp.float32),
            pltpu.VMEM((128,), jnp.int32),
            pltpu.HBM((NUM_CORES * NUM_SUBCORES * MAX_SLOTS * FDIM,), jnp.float32),
            pltpu.SemaphoreType.DMA,
            pltpu.SemaphoreType.DMA,
        ],
    )
    sc_out = sc_run(f, counts).reshape(SC_GRAPHS, FDIM)

    tc_out = pl.pallas_call(
        _tc_body,
        grid=(TC_GRAPHS,),
        in_specs=[
            pl.BlockSpec(memory_space=pltpu.SMEM),
            pl.BlockSpec((ROWS_PER_GRAPH, PATH, DIM), lambda i: (i + SC_GRAPHS, 0, 0)),
        ],
        out_specs=pl.BlockSpec((1, DIM, PATH), lambda i: (i, 0, 0)),
        out_shape=jax.ShapeDtypeStruct((TC_GRAPHS, DIM, PATH), jnp.float32),
    )(n_nodes, features)

    sc_full = jnp.pad(sc_out, ((0, TC_GRAPHS), (0, 0)))
    tc_full = jnp.pad(tc_out.reshape(TC_GRAPHS, FDIM), ((SC_GRAPHS, 0), (0, 0)))
    return sc_full + tc_full


# speedup vs baseline: 1.2028x; 1.2028x over previous
import functools

import jax
import jax.numpy as jnp
from jax import lax
from jax.experimental import pallas as pl
from jax.experimental.pallas import tpu as pltpu
from jax.experimental.pallas import tpu_sc as plsc

N_NODES = 100000
N_GRAPHS = 100
ROWS_PER_GRAPH = 1000
PATH = 4
DIM = 128
FDIM = PATH * DIM
LANES = 16
NCHUNK = FDIM // LANES

CHUNK_ROWS = 100
CHUNK_ELEMS = CHUNK_ROWS * FDIM
CHUNKS_PER_GRAPH = ROWS_PER_GRAPH // CHUNK_ROWS

SC_GRAPHS = 72
TC_GRAPHS = N_GRAPHS - SC_GRAPHS

NUM_CORES = 2
NUM_SUBCORES = 16
G_PER_CORE = SC_GRAPHS // NUM_CORES
CHUNKS_PER_CORE = G_PER_CORE * CHUNKS_PER_GRAPH
MAX_SLOTS = 6


def _span_lo(s):
    return s * CHUNKS_PER_CORE // NUM_SUBCORES


def _owner(c):
    return (c * NUM_SUBCORES + NUM_SUBCORES - 1) // CHUNKS_PER_CORE


def _perm_base(c):
    p = c // (DIM // LANES)
    d0 = (c % (DIM // LANES)) * LANES
    return d0 * PATH + p


def _body(
    features_hbm,
    counts_hbm,
    out_hbm,
    buf0,
    buf1,
    fstage,
    f1,
    f2,
    staging_v,
    counts_v,
    parts_hbm,
    sem0,
    sem1,
):
    cid = lax.axis_index("c")
    sid = lax.axis_index("s")

    pltpu.sync_copy(counts_hbm, counts_v)

    zero = jnp.zeros((LANES,), jnp.float32)
    lane_iota = lax.iota(jnp.int32, LANES)

    c_lo = cid * CHUNKS_PER_CORE + sid * CHUNKS_PER_CORE // NUM_SUBCORES
    c_hi = cid * CHUNKS_PER_CORE + (sid + 1) * CHUNKS_PER_CORE // NUM_SUBCORES
    n_chunks = c_hi - c_lo
    first_graph = (c_lo - cid * CHUNKS_PER_CORE) // CHUNKS_PER_GRAPH
    part_base = (cid * NUM_SUBCORES + sid) * MAX_SLOTS * FDIM

    def chunk_copy(c, buf, sem):
        start = pl.multiple_of((c_lo + c) * CHUNK_ELEMS, 512)
        return pltpu.make_async_copy(
            features_hbm.at[pl.ds(start, CHUNK_ELEMS)], buf, sem
        )

    chunk_copy(0, buf0, sem0).start()
    chunk_copy(1, buf1, sem1).start()

    def step(c, buf, sem, acc):
        chunk_copy(c, buf, sem).wait()

        def row_body(r, a):
            off = r * FDIM
            return tuple(
                a[k] + buf[pl.ds(off + k * LANES, LANES)] for k in range(NCHUNK)
            )

        acc = lax.fori_loop(0, CHUNK_ROWS, row_body, acc)

        gc = c_lo + c
        is_flush = jnp.logical_or((gc + 1) % CHUNKS_PER_GRAPH == 0, c == n_chunks - 1)

        @pl.when(is_flush)
        def _():
            g_local = (gc - cid * CHUNKS_PER_CORE) // CHUNKS_PER_GRAPH
            slot = g_local - first_graph
            for k in range(NCHUNK):
                fstage[pl.ds(k * LANES, LANES)] = acc[k]
            dst = pl.multiple_of(part_base + slot * FDIM, 512)
            pltpu.sync_copy(fstage, parts_hbm.at[pl.ds(dst, FDIM)])

        @pl.when(c + 2 < n_chunks)
        def _():
            chunk_copy(c + 2, buf, sem).start()

        return tuple(jnp.where(is_flush, zero, a) for a in acc)

    def pair_body(i, acc):
        acc = step(2 * i, buf0, sem0, acc)
        acc = step(2 * i + 1, buf1, sem1, acc)
        return acc

    acc = lax.fori_loop(0, n_chunks // 2, pair_body, (zero,) * NCHUNK)

    @pl.when(n_chunks % 2 == 1)
    def _():
        step(n_chunks - 1, buf0, sem0, acc)

    plsc.subcore_barrier()

    gl_lo = sid * G_PER_CORE // NUM_SUBCORES
    gl_hi = (sid + 1) * G_PER_CORE // NUM_SUBCORES
    lane4 = lane_iota * PATH

    def do_out(gl, carry):
        c_first = gl * CHUNKS_PER_GRAPH
        c_last = c_first + CHUNKS_PER_GRAPH - 1
        s1 = _owner(c_first)
        s2 = _owner(c_last)
        slot1 = gl - _span_lo(s1) // CHUNKS_PER_GRAPH
        slot2 = gl - _span_lo(s2) // CHUNKS_PER_GRAPH
        src1 = pl.multiple_of(
            ((cid * NUM_SUBCORES + s1) * MAX_SLOTS + slot1) * FDIM, 512
        )
        src2 = pl.multiple_of(
            ((cid * NUM_SUBCORES + s2) * MAX_SLOTS + slot2) * FDIM, 512
        )
        pltpu.sync_copy(parts_hbm.at[pl.ds(src1, FDIM)], f1)
        pltpu.sync_copy(parts_hbm.at[pl.ds(src2, FDIM)], f2)
        m = jnp.broadcast_to((s1 != s2).astype(jnp.float32), (LANES,))
        g = cid * G_PER_CORE + gl
        cnt = counts_v[pl.ds(g, LANES)][0]
        cnt_v = jnp.broadcast_to(cnt.astype(jnp.float32), (LANES,))
        scale = 1.0 / jnp.maximum(cnt_v, 1.0)
        for k in range(NCHUNK):
            val = f1[pl.ds(k * LANES, LANES)] + m * f2[pl.ds(k * LANES, LANES)]
            plsc.store_scatter(staging_v, [lane4 + _perm_base(k)], val * scale)
        out_start = pl.multiple_of(g * FDIM, 512)
        pltpu.sync_copy(staging_v, out_hbm.at[pl.ds(out_start, FDIM)])
        return carry

    lax.fori_loop(gl_lo, gl_hi, do_out, 0)


def _tc_body(counts_smem, feat_ref, out_ref):
    g = pl.program_id(0) + SC_GRAPHS
    cnt = counts_smem[g]
    scale = 1.0 / jnp.maximum(cnt.astype(jnp.float32), 1.0)
    s = jnp.sum(feat_ref[...], axis=0)
    t = jnp.swapaxes(s, 0, 1)
    out_ref[...] = (t * scale).reshape(1, DIM, PATH)


@jax.jit
def kernel(features, n_nodes):

    f = features.reshape(N_NODES * FDIM)
    counts = jnp.pad(n_nodes, (0, 28))

    sc_run = pl.kernel(
        _body,
        out_type=jax.ShapeDtypeStruct((SC_GRAPHS * FDIM,), jnp.float32),
        mesh=plsc.VectorSubcoreMesh(core_axis_name="c", subcore_axis_name="s"),
        compiler_params=pltpu.CompilerParams(needs_layout_passes=False),
        scratch_types=[
            pltpu.VMEM((CHUNK_ELEMS,), jnp.float32),
            pltpu.VMEM((CHUNK_ELEMS,), jnp.float32),
            pltpu.VMEM((FDIM,), jnp.float32),
            pltpu.VMEM((FDIM,), jnp.float32),
            pltpu.VMEM((FDIM,), jnp.float32),
            pltpu.VMEM((FDIM,), jnp.float32),
            pltpu.VMEM((128,), jnp.int32),
            pltpu.HBM((NUM_CORES * NUM_SUBCORES * MAX_SLOTS * FDIM,), jnp.float32),
            pltpu.SemaphoreType.DMA,
            pltpu.SemaphoreType.DMA,
        ],
    )
    sc_out = sc_run(f, counts).reshape(SC_GRAPHS, FDIM)

    tc_out = pl.pallas_call(
        _tc_body,
        grid=(TC_GRAPHS,),
        in_specs=[
            pl.BlockSpec(memory_space=pltpu.SMEM),
            pl.BlockSpec((ROWS_PER_GRAPH, PATH, DIM), lambda i: (i + SC_GRAPHS, 0, 0)),
        ],
        out_specs=pl.BlockSpec((1, DIM, PATH), lambda i: (i, 0, 0)),
        out_shape=jax.ShapeDtypeStruct((TC_GRAPHS, DIM, PATH), jnp.float32),
    )(n_nodes, features)

    sc_full = jnp.pad(sc_out, ((0, TC_GRAPHS), (0, 0)))
    tc_full = jnp.pad(tc_out.reshape(TC_GRAPHS, FDIM), ((SC_GRAPHS, 0), (0, 0)))
    return sc_full + tc_full


# speedup vs baseline: 1.2665x; 1.0530x over previous
import functools

import jax
import jax.numpy as jnp
from jax import lax
from jax.experimental import pallas as pl
from jax.experimental.pallas import tpu as pltpu
from jax.experimental.pallas import tpu_sc as plsc

N_NODES = 100000
N_GRAPHS = 100
ROWS_PER_GRAPH = 1000
PATH = 4
DIM = 128
FDIM = PATH * DIM
LANES = 16
NCHUNK = FDIM // LANES

CHUNK_ROWS = 100
CHUNK_ELEMS = CHUNK_ROWS * FDIM
CHUNKS_PER_GRAPH = ROWS_PER_GRAPH // CHUNK_ROWS

SC_GRAPHS = 64
TC_GRAPHS = N_GRAPHS - SC_GRAPHS

NUM_CORES = 2
NUM_SUBCORES = 16
G_PER_CORE = SC_GRAPHS // NUM_CORES
CHUNKS_PER_CORE = G_PER_CORE * CHUNKS_PER_GRAPH
MAX_SLOTS = 6


def _span_lo(s):
    return s * CHUNKS_PER_CORE // NUM_SUBCORES


def _owner(c):
    return (c * NUM_SUBCORES + NUM_SUBCORES - 1) // CHUNKS_PER_CORE


def _perm_base(c):
    p = c // (DIM // LANES)
    d0 = (c % (DIM // LANES)) * LANES
    return d0 * PATH + p


def _body(
    features_hbm,
    counts_hbm,
    out_hbm,
    buf0,
    buf1,
    fstage,
    f1,
    f2,
    staging_v,
    counts_v,
    parts_hbm,
    sem0,
    sem1,
):
    cid = lax.axis_index("c")
    sid = lax.axis_index("s")

    pltpu.sync_copy(counts_hbm, counts_v)

    zero = jnp.zeros((LANES,), jnp.float32)
    lane_iota = lax.iota(jnp.int32, LANES)

    c_lo = cid * CHUNKS_PER_CORE + sid * CHUNKS_PER_CORE // NUM_SUBCORES
    c_hi = cid * CHUNKS_PER_CORE + (sid + 1) * CHUNKS_PER_CORE // NUM_SUBCORES
    n_chunks = c_hi - c_lo
    first_graph = (c_lo - cid * CHUNKS_PER_CORE) // CHUNKS_PER_GRAPH
    part_base = (cid * NUM_SUBCORES + sid) * MAX_SLOTS * FDIM

    def chunk_copy(c, buf, sem):
        start = pl.multiple_of((c_lo + c) * CHUNK_ELEMS, 512)
        return pltpu.make_async_copy(
            features_hbm.at[pl.ds(start, CHUNK_ELEMS)], buf, sem
        )

    chunk_copy(0, buf0, sem0).start()
    chunk_copy(1, buf1, sem1).start()

    def step(c, buf, sem, acc):
        chunk_copy(c, buf, sem).wait()

        def row_body(r, a):
            off = r * FDIM
            return tuple(
                a[k] + buf[pl.ds(off + k * LANES, LANES)] for k in range(NCHUNK)
            )

        acc = lax.fori_loop(0, CHUNK_ROWS, row_body, acc)

        gc = c_lo + c
        is_flush = jnp.logical_or((gc + 1) % CHUNKS_PER_GRAPH == 0, c == n_chunks - 1)

        @pl.when(is_flush)
        def _():
            g_local = (gc - cid * CHUNKS_PER_CORE) // CHUNKS_PER_GRAPH
            slot = g_local - first_graph
            for k in range(NCHUNK):
                fstage[pl.ds(k * LANES, LANES)] = acc[k]
            dst = pl.multiple_of(part_base + slot * FDIM, 512)
            pltpu.sync_copy(fstage, parts_hbm.at[pl.ds(dst, FDIM)])

        @pl.when(c + 2 < n_chunks)
        def _():
            chunk_copy(c + 2, buf, sem).start()

        return tuple(jnp.where(is_flush, zero, a) for a in acc)

    def pair_body(i, acc):
        acc = step(2 * i, buf0, sem0, acc)
        acc = step(2 * i + 1, buf1, sem1, acc)
        return acc

    acc = lax.fori_loop(0, n_chunks // 2, pair_body, (zero,) * NCHUNK)

    @pl.when(n_chunks % 2 == 1)
    def _():
        step(n_chunks - 1, buf0, sem0, acc)

    plsc.subcore_barrier()

    gl_lo = sid * G_PER_CORE // NUM_SUBCORES
    gl_hi = (sid + 1) * G_PER_CORE // NUM_SUBCORES
    lane4 = lane_iota * PATH

    def do_out(gl, carry):
        c_first = gl * CHUNKS_PER_GRAPH
        c_last = c_first + CHUNKS_PER_GRAPH - 1
        s1 = _owner(c_first)
        s2 = _owner(c_last)
        slot1 = gl - _span_lo(s1) // CHUNKS_PER_GRAPH
        slot2 = gl - _span_lo(s2) // CHUNKS_PER_GRAPH
        src1 = pl.multiple_of(
            ((cid * NUM_SUBCORES + s1) * MAX_SLOTS + slot1) * FDIM, 512
        )
        src2 = pl.multiple_of(
            ((cid * NUM_SUBCORES + s2) * MAX_SLOTS + slot2) * FDIM, 512
        )
        pltpu.sync_copy(parts_hbm.at[pl.ds(src1, FDIM)], f1)
        pltpu.sync_copy(parts_hbm.at[pl.ds(src2, FDIM)], f2)
        m = jnp.broadcast_to((s1 != s2).astype(jnp.float32), (LANES,))
        g = cid * G_PER_CORE + gl
        cnt = counts_v[pl.ds(g, LANES)][0]
        cnt_v = jnp.broadcast_to(cnt.astype(jnp.float32), (LANES,))
        scale = 1.0 / jnp.maximum(cnt_v, 1.0)
        for k in range(NCHUNK):
            val = f1[pl.ds(k * LANES, LANES)] + m * f2[pl.ds(k * LANES, LANES)]
            plsc.store_scatter(staging_v, [lane4 + _perm_base(k)], val * scale)
        out_start = pl.multiple_of(g * FDIM, 512)
        pltpu.sync_copy(staging_v, out_hbm.at[pl.ds(out_start, FDIM)])
        return carry

    lax.fori_loop(gl_lo, gl_hi, do_out, 0)


def _tc_body(counts_smem, feat_ref, out_ref):
    g = pl.program_id(0) + SC_GRAPHS
    cnt = counts_smem[g]
    scale = 1.0 / jnp.maximum(cnt.astype(jnp.float32), 1.0)
    s = jnp.sum(feat_ref[...], axis=0)
    t = jnp.swapaxes(s, 0, 1)
    out_ref[...] = (t * scale).reshape(1, DIM, PATH)


@jax.jit
def kernel(features, n_nodes):

    f = features.reshape(N_NODES * FDIM)
    counts = jnp.pad(n_nodes, (0, 28))

    sc_run = pl.kernel(
        _body,
        out_type=jax.ShapeDtypeStruct((SC_GRAPHS * FDIM,), jnp.float32),
        mesh=plsc.VectorSubcoreMesh(core_axis_name="c", subcore_axis_name="s"),
        compiler_params=pltpu.CompilerParams(needs_layout_passes=False),
        scratch_types=[
            pltpu.VMEM((CHUNK_ELEMS,), jnp.float32),
            pltpu.VMEM((CHUNK_ELEMS,), jnp.float32),
            pltpu.VMEM((FDIM,), jnp.float32),
            pltpu.VMEM((FDIM,), jnp.float32),
            pltpu.VMEM((FDIM,), jnp.float32),
            pltpu.VMEM((FDIM,), jnp.float32),
            pltpu.VMEM((128,), jnp.int32),
            pltpu.HBM((NUM_CORES * NUM_SUBCORES * MAX_SLOTS * FDIM,), jnp.float32),
            pltpu.SemaphoreType.DMA,
            pltpu.SemaphoreType.DMA,
        ],
    )
    sc_out = sc_run(f, counts).reshape(SC_GRAPHS, FDIM)

    tc_out = pl.pallas_call(
        _tc_body,
        grid=(TC_GRAPHS,),
        in_specs=[
            pl.BlockSpec(memory_space=pltpu.SMEM),
            pl.BlockSpec((ROWS_PER_GRAPH, PATH, DIM), lambda i: (i + SC_GRAPHS, 0, 0)),
        ],
        out_specs=pl.BlockSpec((1, DIM, PATH), lambda i: (i, 0, 0)),
        out_shape=jax.ShapeDtypeStruct((TC_GRAPHS, DIM, PATH), jnp.float32),
    )(n_nodes, features)

    sc_full = jnp.pad(sc_out, ((0, TC_GRAPHS), (0, 0)))
    tc_full = jnp.pad(tc_out.reshape(TC_GRAPHS, FDIM), ((SC_GRAPHS, 0), (0, 0)))
    return sc_full + tc_full
